# Initial kernel scaffold; baseline (speedup 1.0000x reference)
import functools

import jax
import jax.numpy as jnp
from jax import lax
from jax.experimental import pallas as pl
from jax.experimental.pallas import tpu as pltpu
from jax.experimental.pallas import tpu_sc as plsc

N, S, H, W, D = 8, 32, 14, 14, 8
E = 64
HWD = H * W * D
ROW_WORDS = HWD * E
NC, NS = 2, 16


def _sc_body(table_hbm, out_hbm, buf, sem):
    s = lax.axis_index("s") * NC + lax.axis_index("c")

    pltpu.sync_copy(table_hbm.at[s], buf.at[pl.ds(0, E)])

    v0 = buf[pl.ds(0, 16)]
    v1 = buf[pl.ds(16, 16)]
    v2 = buf[pl.ds(32, 16)]
    v3 = buf[pl.ds(48, 16)]

    def rep(i, carry):
        off = pl.multiple_of(i * E, E)
        buf[pl.ds(off, 16)] = v0
        buf[pl.ds(off + 16, 16)] = v1
        buf[pl.ds(off + 32, 16)] = v2
        buf[pl.ds(off + 48, 16)] = v3
        return carry

    lax.fori_loop(1, HWD, rep, 0)

    copies = [pltpu.async_copy(buf, out_hbm.at[n * S + s], sem) for n in range(N)]
    for c in copies:
        c.wait()


@jax.jit
def _sc_embed(table):
    mesh = plsc.VectorSubcoreMesh(core_axis_name="c", subcore_axis_name="s")
    k = pl.kernel(
        _sc_body,
        mesh=mesh,
        out_type=jax.ShapeDtypeStruct((N * S, ROW_WORDS), jnp.float32),
        scratch_types=[
            pltpu.VMEM((ROW_WORDS,), jnp.float32),
            pltpu.SemaphoreType.DMA,
        ],
    )
    return k(table)


def kernel(x, table):
    n, s, h, w, d = x.shape
    out = _sc_embed(table)
    return out.reshape(n, s, h, w, d, E)


# speedup vs baseline: 7.0478x; 7.0478x over previous
import functools

import jax
import jax.numpy as jnp
from jax import lax
from jax.experimental import pallas as pl
from jax.experimental.pallas import tpu as pltpu
from jax.experimental.pallas import tpu_sc as plsc

N, S, H, W, D = 8, 32, 14, 14, 8
E = 64
HWD = H * W * D
ROW_WORDS = HWD * E
NC, NS = 2, 16


def _sc_body(table_hbm, out_hbm, buf, sem):
    s = lax.axis_index("s") * NC + lax.axis_index("c")

    pltpu.sync_copy(table_hbm.at[pl.ds(s * E, E)], buf.at[pl.ds(0, E)])

    v0 = buf[pl.ds(0, 16)]
    v1 = buf[pl.ds(16, 16)]
    v2 = buf[pl.ds(32, 16)]
    v3 = buf[pl.ds(48, 16)]

    def rep(i, carry):
        off = pl.multiple_of(i * E, E)
        buf[pl.ds(off, 16)] = v0
        buf[pl.ds(off + 16, 16)] = v1
        buf[pl.ds(off + 32, 16)] = v2
        buf[pl.ds(off + 48, 16)] = v3
        return carry

    lax.fori_loop(1, HWD, rep, 0)

    copies = [
        pltpu.async_copy(
            buf, out_hbm.at[pl.ds((n * S + s) * ROW_WORDS, ROW_WORDS)], sem
        )
        for n in range(N)
    ]
    for c in copies:
        c.wait()


@jax.jit
def _sc_embed(table):
    mesh = plsc.VectorSubcoreMesh(core_axis_name="c", subcore_axis_name="s")
    k = pl.kernel(
        _sc_body,
        mesh=mesh,
        out_type=jax.ShapeDtypeStruct((N * S * ROW_WORDS,), jnp.float32),
        scratch_types=[
            pltpu.VMEM((ROW_WORDS,), jnp.float32),
            pltpu.SemaphoreType.DMA,
        ],
    )
    return k(table.reshape(-1))


def kernel(x, table):
    n, s, h, w, d = x.shape
    out = _sc_embed(table)
    return out.reshape(n, s, h, w, d, E)


# speedup vs baseline: 17.7127x; 2.5132x over previous
import jax
import jax.numpy as jnp
from jax import lax
from jax.experimental import pallas as pl
from jax.experimental.pallas import tpu as pltpu
from jax.experimental.pallas import tpu_sc as plsc

N, S, H, W, D = 8, 32, 14, 14, 8
E = 64
NC, NS = 2, 16


def _sc_body(table_hbm, out_hbm, buf, sem):
    s = lax.axis_index("s") * NC + lax.axis_index("c")

    pltpu.sync_copy(table_hbm.at[pl.ds(s * E, E)], buf.at[0, 0, 0])

    regs = [buf[0, 0, 0, pl.ds(16 * j, 16)] for j in range(4)]

    def rep(i, carry):
        h = i // W
        w = lax.rem(i, W)
        for d in range(D):
            for j in range(4):
                buf[h, w, d, pl.ds(16 * j, 16)] = regs[j]
        return carry

    lax.fori_loop(0, (H // 2) * W, rep, 0)

    copies = [
        pltpu.async_copy(buf, out_hbm.at[n, s, pl.ds(half * (H // 2), H // 2)], sem)
        for n in range(N)
        for half in range(2)
    ]
    for c in copies:
        c.wait()


@jax.jit
def _sc_embed(table):
    mesh = plsc.VectorSubcoreMesh(core_axis_name="c", subcore_axis_name="s")
    k = pl.kernel(
        _sc_body,
        mesh=mesh,
        out_type=jax.ShapeDtypeStruct((N, S, H, W, D, E), jnp.float32),
        scratch_types=[
            pltpu.VMEM((H // 2, W, D, E), jnp.float32),
            pltpu.SemaphoreType.DMA,
        ],
    )
    return k(table.reshape(-1))


def kernel(x, table):
    del x
    return _sc_embed(table)
